# baseline (speedup 1.0000x reference)
import jax
import jax.numpy as jnp
import numpy as np
from jax.experimental import pallas as pl
from jax.experimental.pallas import tpu as pltpu
from jax.sharding import Mesh, PartitionSpec as P

_E = 8
_H = 2048
_F = 2048
_ALPHA = 1.702
_LIMIT = 7.0

_FT = 256
_NFT = _F // _FT
_HT = 256
_NHT = _H // _HT
assert _NHT == _NFT


def _make_body(tl):
    def _moe_body(hs_ref, rw_ref, gub_ref, dnb_ref, guw_ref, dnw_ref, out_ref,
                  fused_ref):
        e = pl.program_id(0)
        j = pl.program_id(1)

        @pl.when((e == 0) & (j == 0))
        def _init():
            out_ref[...] = jnp.dot(rw_ref[...], dnb_ref[...],
                                   preferred_element_type=jnp.float32)

        @pl.when(e < _E)
        def _stage_a():
            w = guw_ref[0].astype(jnp.bfloat16)
            gup = jnp.dot(hs_ref[...], w, preferred_element_type=jnp.float32)
            b = gub_ref[pl.ds(e, 1), pl.ds(j * (2 * _FT), 2 * _FT)]
            gup = (gup + b).astype(jnp.bfloat16)

            r = jax.lax.broadcasted_iota(jnp.int32, (2 * _FT, _FT), 0)
            c = jax.lax.broadcasted_iota(jnp.int32, (2 * _FT, _FT), 1)
            sel_gate = (r == 2 * c).astype(jnp.bfloat16)
            sel_up = (r == 2 * c + 1).astype(jnp.bfloat16)
            gate = jnp.dot(gup, sel_gate, preferred_element_type=jnp.float32)
            up = jnp.dot(gup, sel_up, preferred_element_type=jnp.float32)

            gate = jnp.minimum(gate, _LIMIT)
            up = jnp.clip(up, -_LIMIT, _LIMIT)
            glu = gate * jax.nn.sigmoid(gate * _ALPHA)
            lane = jax.lax.broadcasted_iota(jnp.int32, (tl, _E), 1)
            rwc = jnp.sum(jnp.where(lane == e, rw_ref[...], 0.0), axis=1,
                          keepdims=True)
            fused = (up + 1.0) * glu * rwc
            fused_ref[e % 2, :, pl.ds(j * _FT, _FT)] = fused.astype(
                jnp.bfloat16)

        @pl.when(e > 0)
        def _stage_b():
            dw = dnw_ref[0].astype(jnp.bfloat16)
            tile = jnp.dot(fused_ref[(e - 1) % 2], dw,
                           preferred_element_type=jnp.float32)
            out_ref[:, pl.ds(j * _HT, _HT)] += tile

    return _moe_body


def _moe_local(hs, rw, gub, dnb, guw, dnw):
    tl = hs.shape[0]
    return pl.pallas_call(
        _make_body(tl),
        grid=(_E + 1, _NFT),
        in_specs=[
            pl.BlockSpec((tl, _H), lambda e, j: (0, 0)),
            pl.BlockSpec((tl, _E), lambda e, j: (0, 0)),
            pl.BlockSpec((_E, 2 * _F), lambda e, j: (0, 0)),
            pl.BlockSpec((_E, _H), lambda e, j: (0, 0)),
            pl.BlockSpec((1, _H, 2 * _FT),
                         lambda e, j: (jnp.minimum(e, _E - 1), 0, j)),
            pl.BlockSpec((1, _F, _HT),
                         lambda e, j: (jnp.maximum(e - 1, 0), 0, j)),
        ],
        out_specs=pl.BlockSpec((tl, _H), lambda e, j: (0, 0)),
        out_shape=jax.ShapeDtypeStruct((tl, _H), jnp.float32),
        scratch_shapes=[pltpu.VMEM((2, tl, _F), jnp.bfloat16)],
        compiler_params=pltpu.CompilerParams(
            dimension_semantics=("arbitrary", "arbitrary"),
            vmem_limit_bytes=64 * 1024 * 1024,
        ),
    )(hs, rw, gub, dnb, guw, dnw)


def kernel(hidden_states, router_indices, routing_weights, gate_up_w,
           gate_up_b, down_w, down_b):
    del router_indices
    batch = hidden_states.shape[0]
    hs = hidden_states.reshape(-1, _H).astype(jnp.bfloat16)

    mesh = Mesh(np.array(jax.devices()[:2]), ("x",))
    sharded = jax.shard_map(
        _moe_local,
        mesh=mesh,
        in_specs=(P("x"), P("x"), P(), P(), P(), P()),
        out_specs=P("x"),
        check_vma=False,
    )
    out = sharded(hs, routing_weights, gate_up_b, down_b, gate_up_w, down_w)
    return out.reshape(batch, -1, _H)


# speedup vs baseline: 1.9419x; 1.9419x over previous
import jax
import jax.numpy as jnp
from jax.experimental import pallas as pl
from jax.experimental.pallas import tpu as pltpu

_E = 8
_H = 2048
_F = 2048
_T = 2048
_ALPHA = 1.702
_LIMIT = 7.0

_FT = 256
_NFT = _F // _FT
_HT = 512
_NHT = _H // _HT


def _moe_body(hs_ref, rw_ref, gub_ref, dnb_ref, guw_ref, dnw_ref, out_ref,
              fused_ref):
    e = pl.program_id(0)
    j = pl.program_id(1)

    @pl.when((e == 0) & (j == 0))
    def _init():
        out_ref[...] = jnp.dot(rw_ref[...], dnb_ref[...],
                               preferred_element_type=jnp.float32)

    @pl.when(e < _E)
    def _stage_a():
        w = guw_ref[0].astype(jnp.bfloat16)
        gup = jnp.dot(hs_ref[...], w, preferred_element_type=jnp.float32)
        b = gub_ref[pl.ds(e, 1), pl.ds(j * (2 * _FT), 2 * _FT)]
        gup = (gup + b).astype(jnp.bfloat16)

        r = jax.lax.broadcasted_iota(jnp.int32, (2 * _FT, _FT), 0)
        c = jax.lax.broadcasted_iota(jnp.int32, (2 * _FT, _FT), 1)
        sel_gate = (r == 2 * c).astype(jnp.bfloat16)
        sel_up = (r == 2 * c + 1).astype(jnp.bfloat16)
        gate = jnp.dot(gup, sel_gate, preferred_element_type=jnp.float32)
        up = jnp.dot(gup, sel_up, preferred_element_type=jnp.float32)

        gate = jnp.minimum(gate, _LIMIT)
        up = jnp.clip(up, -_LIMIT, _LIMIT)
        glu = gate * jax.nn.sigmoid(gate * _ALPHA)
        lane = jax.lax.broadcasted_iota(jnp.int32, (_T, _E), 1)
        rwc = jnp.sum(jnp.where(lane == e, rw_ref[...], 0.0), axis=1,
                      keepdims=True)
        fused = (up + 1.0) * glu * rwc
        fused_ref[e % 2, :, pl.ds(j * _FT, _FT)] = fused.astype(jnp.bfloat16)

    @pl.when((e > 0) & (j < _NHT))
    def _stage_b():
        dw = dnw_ref[0].astype(jnp.bfloat16)
        tile = jnp.dot(fused_ref[(e - 1) % 2], dw,
                       preferred_element_type=jnp.float32)
        out_ref[:, pl.ds(j * _HT, _HT)] += tile


def kernel(hidden_states, router_indices, routing_weights, gate_up_w,
           gate_up_b, down_w, down_b):
    del router_indices
    batch = hidden_states.shape[0]
    hs = hidden_states.reshape(-1, _H).astype(jnp.bfloat16)

    out = pl.pallas_call(
        _moe_body,
        grid=(_E + 1, _NFT),
        in_specs=[
            pl.BlockSpec((_T, _H), lambda e, j: (0, 0)),
            pl.BlockSpec((_T, _E), lambda e, j: (0, 0)),
            pl.BlockSpec((_E, 2 * _F), lambda e, j: (0, 0)),
            pl.BlockSpec((_E, _H), lambda e, j: (0, 0)),
            pl.BlockSpec((1, _H, 2 * _FT),
                         lambda e, j: (jnp.minimum(e, _E - 1), 0, j)),
            pl.BlockSpec((1, _F, _HT),
                         lambda e, j: (jnp.maximum(e - 1, 0), 0,
                                       jnp.minimum(j, _NHT - 1))),
        ],
        out_specs=pl.BlockSpec((_T, _H), lambda e, j: (0, 0)),
        out_shape=jax.ShapeDtypeStruct((_T, _H), jnp.float32),
        scratch_shapes=[pltpu.VMEM((2, _T, _F), jnp.bfloat16)],
        compiler_params=pltpu.CompilerParams(
            dimension_semantics=("arbitrary", "arbitrary"),
            vmem_limit_bytes=64 * 1024 * 1024,
        ),
    )(hs, routing_weights, gate_up_b, down_b, gate_up_w, down_w)

    return out.reshape(batch, -1, _H)
